# baseline (speedup 1.0000x reference)
import jax
import jax.numpy as jnp
from jax import lax
from jax.experimental import pallas as pl
from jax.experimental.pallas import tpu as pltpu
from jax.experimental.pallas import tpu_sc as plsc

NC = 2
NS = 16
LANES = 16
NW = NC * NS


def _mesh():
    return plsc.VectorSubcoreMesh(
        core_axis_name="c", subcore_axis_name="s", num_cores=NC, num_subcores=NS
    )


def _sc_gather(table, idx):
    V, D = table.shape
    B = idx.shape[0]
    rows_per = B // NW

    def body(tab_hbm, idx_hbm, out_hbm, idx_v, rows_v, sem):
        cid = lax.axis_index("c")
        sid = lax.axis_index("s")
        wid = cid * NS + sid
        base = wid * rows_per
        pltpu.sync_copy(idx_hbm.at[pl.ds(base, rows_per)], idx_v)
        pltpu.async_copy(tab_hbm.at[idx_v], rows_v, sem).wait()
        pltpu.sync_copy(rows_v, out_hbm.at[pl.ds(base, rows_per)])

    k = pl.kernel(
        body,
        out_type=jax.ShapeDtypeStruct((B, D), table.dtype),
        mesh=_mesh(),
        scratch_types=[
            pltpu.VMEM((rows_per,), jnp.int32),
            pltpu.VMEM((rows_per, D), jnp.float32),
            pltpu.SemaphoreType.DMA,
        ],
    )
    return k(table, idx)


def _sc_gather2(h, c, hi):
    Lv, D = h.shape
    B = hi.shape[0]
    rows_per = B // NW
    GCH = 64

    def body(h_hbm, c_hbm, hi_hbm, hch_hbm, cch_hbm, idx_v, bufa_v, bufb_v,
             sema, semb):
        cid = lax.axis_index("c")
        sid = lax.axis_index("s")
        base = (cid * NS + sid) * rows_per
        wa = wb = None
        for k in range(rows_per // GCH):
            b = base + k * GCH
            pltpu.sync_copy(hi_hbm.at[pl.ds(b, GCH)], idx_v)
            if wa is not None:
                wa.wait()
            ga = pltpu.async_copy(h_hbm.at[idx_v], bufa_v, sema)
            if wb is not None:
                wb.wait()
            gb = pltpu.async_copy(c_hbm.at[idx_v], bufb_v, semb)
            ga.wait()
            wa = pltpu.async_copy(bufa_v, hch_hbm.at[pl.ds(b, GCH)], sema)
            gb.wait()
            wb = pltpu.async_copy(bufb_v, cch_hbm.at[pl.ds(b, GCH)], semb)
        wa.wait()
        wb.wait()

    k = pl.kernel(
        body,
        out_type=[
            jax.ShapeDtypeStruct((B, D), jnp.float32),
            jax.ShapeDtypeStruct((B, D), jnp.float32),
        ],
        mesh=_mesh(),
        scratch_types=[
            pltpu.VMEM((GCH,), jnp.int32),
            pltpu.VMEM((GCH, D), jnp.float32),
            pltpu.VMEM((GCH, D), jnp.float32),
            pltpu.SemaphoreType.DMA,
            pltpu.SemaphoreType.DMA,
        ],
    )
    return k(h, c, hi)


def _sc_segsum(rows, ti, bounds, Lv):
    B, D = rows.shape
    NQ = 8
    Q = Lv // NQ
    CG = D // 4
    ACH = 320

    def body(rows_hbm, ti_hbm, bnd_hbm, out_hbm, acc_v, colbuf0_v, colbuf1_v,
             ti0_v, ti1_v, bnd_v, sem0, sem1):
        cid = lax.axis_index("c")
        sid = lax.axis_index("s")
        wid = cid * NS + sid
        pg = wid // 4
        cg = wid % 4
        pltpu.sync_copy(bnd_hbm.at[pl.ds(wid * LANES, LANES)], bnd_v)
        bvec = bnd_v[...]
        lo = bvec[0]
        hi = bvec[1]
        lo8 = (lo // 8) * 8
        nck = jnp.maximum(hi - lo8 + ACH - 1, 0) // ACH
        nck2 = (nck + 1) // 2
        NCB = CG // LANES

        def chunk_start(ck):
            return jnp.minimum(lo8 + ck * ACH, B - ACH)

        def issue(ck, colbuf, tibuf, sem):
            s = chunk_start(ck)
            pltpu.async_copy(
                rows_hbm.at[pl.ds(s, ACH), pl.ds(cg * CG, CG)], colbuf, sem
            )
            pltpu.async_copy(ti_hbm.at[pl.ds(s, ACH)], tibuf, sem)

        def drain(ck, colbuf, tibuf, sem):
            s = chunk_start(ck)
            pltpu.make_async_copy(
                rows_hbm.at[pl.ds(s, ACH), pl.ds(cg * CG, CG)], colbuf, sem
            ).wait()
            pltpu.make_async_copy(ti_hbm.at[pl.ds(s, ACH)], tibuf, sem).wait()

        @pl.when(nck2 > 0)
        def _():
            issue(0, colbuf0_v, ti0_v, sem0)
            issue(1, colbuf1_v, ti1_v, sem1)

        @pl.loop(0, Q + 1)
        def _(i):
            for cb in range(NCB):
                acc_v[pl.ds(i, 1), pl.ds(cb * LANES, LANES)] = jnp.zeros(
                    (1, LANES), jnp.float32
                )

        def scan_chunk(ck, colbuf_v, ti_v):
            s = chunk_start(ck)
            s_nom = lo8 + ck * ACH
            jlo = jnp.maximum(lo, s_nom)

            @pl.loop(0, ACH // LANES)
            def _(jg):
                tvec = ti_v[pl.ds(jg * LANES, LANES)]
                for l in range(LANES):
                    j = jg * LANES + l
                    r = s + j
                    valid = (r >= jlo) & (r < hi)
                    p = jnp.where(valid, tvec[l] - pg * Q, Q)
                    for cb in range(NCB):
                        sl = (pl.ds(p, 1), pl.ds(cb * LANES, LANES))
                        acc_v[sl] = acc_v[sl] + colbuf_v[
                            pl.ds(j, 1), pl.ds(cb * LANES, LANES)
                        ]

        @pl.loop(0, nck2)
        def _(ck2):
            ck = 2 * ck2
            drain(ck, colbuf0_v, ti0_v, sem0)
            scan_chunk(ck, colbuf0_v, ti0_v)

            @pl.when(ck + 2 < 2 * nck2)
            def _():
                issue(ck + 2, colbuf0_v, ti0_v, sem0)

            drain(ck + 1, colbuf1_v, ti1_v, sem1)
            scan_chunk(ck + 1, colbuf1_v, ti1_v)

            @pl.when(ck + 3 < 2 * nck2)
            def _():
                issue(ck + 3, colbuf1_v, ti1_v, sem1)

        pltpu.sync_copy(
            acc_v.at[pl.ds(0, Q)],
            out_hbm.at[pl.ds(pg * Q, Q), pl.ds(cg * CG, CG)]
        )

    k = pl.kernel(
        body,
        out_type=jax.ShapeDtypeStruct((Lv, D), jnp.float32),
        mesh=_mesh(),
        scratch_types=[
            pltpu.VMEM((Q + 1, CG), jnp.float32),
            pltpu.VMEM((ACH, CG), jnp.float32),
            pltpu.VMEM((ACH, CG), jnp.float32),
            pltpu.VMEM((ACH,), jnp.int32),
            pltpu.VMEM((ACH,), jnp.int32),
            pltpu.VMEM((LANES,), jnp.int32),
            pltpu.SemaphoreType.DMA,
            pltpu.SemaphoreType.DMA,
        ],
    )
    return k(rows, ti, bounds)


def _seg_bounds(ti, Lv):
    edges = jnp.arange(0, Lv + 1, Lv // 8, dtype=jnp.int32)
    b = jnp.searchsorted(ti, edges).astype(jnp.int32)
    pgv = jnp.arange(NW, dtype=jnp.int32) // 4
    row = jnp.zeros((NW, LANES), jnp.int32)
    row = row.at[:, 0].set(b[pgv]).at[:, 1].set(b[pgv + 1])
    return row.reshape(-1)


def _tc_leaf(x0, W_iou, b_iou):
    M, K = x0.shape
    N = W_iou.shape[1]
    H = N // 3
    BM = 512

    def body(x_ref, w_ref, b_ref, h_ref, c_ref):
        iou = jnp.dot(x_ref[...].astype(jnp.bfloat16),
                      w_ref[...].astype(jnp.bfloat16),
                      preferred_element_type=jnp.float32)
        iou = iou + b_ref[...]
        i = jax.nn.sigmoid(iou[:, :H])
        o = jax.nn.sigmoid(iou[:, H:2 * H])
        u = jnp.tanh(iou[:, 2 * H:])
        cc = i * u
        c_ref[...] = cc
        h_ref[...] = o * jnp.tanh(cc)

    return pl.pallas_call(
        body,
        grid=(M // BM,),
        in_specs=[
            pl.BlockSpec((BM, K), lambda i: (i, 0)),
            pl.BlockSpec((K, N), lambda i: (0, 0)),
            pl.BlockSpec((1, N), lambda i: (0, 0)),
        ],
        out_specs=[
            pl.BlockSpec((BM, H), lambda i: (i, 0)),
            pl.BlockSpec((BM, H), lambda i: (i, 0)),
        ],
        out_shape=[
            jax.ShapeDtypeStruct((M, H), jnp.float32),
            jax.ShapeDtypeStruct((M, H), jnp.float32),
        ],
    )(x0, W_iou, b_iou.reshape(1, N))


def _tc_iou(hsum, U_iou, b_iou):
    M, K = hsum.shape
    N = U_iou.shape[1]
    H = N // 3
    BM = 512

    def body(hs_ref, uiou_ref, biou_ref, iu_ref, o_ref):
        hs = hs_ref[...].astype(jnp.bfloat16)
        iou = jnp.dot(hs, uiou_ref[...].astype(jnp.bfloat16),
                      preferred_element_type=jnp.float32)
        iou = iou + biou_ref[...]
        i = jax.nn.sigmoid(iou[:, :H])
        o = jax.nn.sigmoid(iou[:, H:2 * H])
        u = jnp.tanh(iou[:, 2 * H:])
        iu_ref[...] = i * u
        o_ref[...] = o

    return pl.pallas_call(
        body,
        grid=(M // BM,),
        in_specs=[
            pl.BlockSpec((BM, K), lambda i: (i, 0)),
            pl.BlockSpec((K, N), lambda i: (0, 0)),
            pl.BlockSpec((1, N), lambda i: (0, 0)),
        ],
        out_specs=[
            pl.BlockSpec((BM, H), lambda i: (i, 0)),
            pl.BlockSpec((BM, H), lambda i: (i, 0)),
        ],
        out_shape=[
            jax.ShapeDtypeStruct((M, H), jnp.float32),
            jax.ShapeDtypeStruct((M, H), jnp.float32),
        ],
    )(hsum, U_iou, b_iou.reshape(1, N))


def _tc_f(h_ch, c_ch, U_f, b_f):
    B, K = h_ch.shape
    H = U_f.shape[1]
    BM = 512

    def body(hch_ref, cch_ref, uf_ref, bf_ref, g_ref):
        f = jax.nn.sigmoid(
            jnp.dot(hch_ref[...].astype(jnp.bfloat16),
                    uf_ref[...].astype(jnp.bfloat16),
                    preferred_element_type=jnp.float32)
            + bf_ref[...]
        )
        g_ref[...] = f * cch_ref[...]

    return pl.pallas_call(
        body,
        grid=(B // BM,),
        in_specs=[
            pl.BlockSpec((BM, K), lambda i: (i, 0)),
            pl.BlockSpec((BM, K), lambda i: (i, 0)),
            pl.BlockSpec((K, H), lambda i: (0, 0)),
            pl.BlockSpec((1, H), lambda i: (0, 0)),
        ],
        out_specs=pl.BlockSpec((BM, H), lambda i: (i, 0)),
        out_shape=jax.ShapeDtypeStruct((B, H), jnp.float32),
    )(h_ch, c_ch, U_f, b_f.reshape(1, H))


def _tc_final(iu, o, fc):
    M, H = iu.shape
    BM = 1024

    def body(iu_ref, o_ref, fc_ref, h_ref, c_ref):
        c = iu_ref[...] + fc_ref[...]
        c_ref[...] = c
        h_ref[...] = o_ref[...] * jnp.tanh(c)

    return pl.pallas_call(
        body,
        grid=(M // BM,),
        in_specs=[
            pl.BlockSpec((BM, H), lambda i: (i, 0)),
            pl.BlockSpec((BM, H), lambda i: (i, 0)),
            pl.BlockSpec((BM, H), lambda i: (i, 0)),
        ],
        out_specs=[
            pl.BlockSpec((BM, H), lambda i: (i, 0)),
            pl.BlockSpec((BM, H), lambda i: (i, 0)),
        ],
        out_shape=[
            jax.ShapeDtypeStruct((M, H), jnp.float32),
            jax.ShapeDtypeStruct((M, H), jnp.float32),
        ],
    )(iu, o, fc)


def kernel(x, word_idx, hidden_idx, tree_idx, W_iou, U_iou, b_iou, W_f, U_f, b_f):
    L = word_idx.shape[0]
    x0 = _sc_gather(x, word_idx)
    h, c = _tc_leaf(x0, W_iou, b_iou)
    for d in range(hidden_idx.shape[0]):
        hi = hidden_idx[d]
        ti = tree_idx[d]
        bounds = _seg_bounds(ti, L)
        h_ch, c_ch = _sc_gather2(h, c, hi)
        g = _tc_f(h_ch, c_ch, U_f, b_f)
        hsum = _sc_segsum(h_ch, ti, bounds, L)
        fc = _sc_segsum(g, ti, bounds, L)
        iu, o = _tc_iou(hsum, U_iou, b_iou)
        h, c = _tc_final(iu, o, fc)
    return (h, c)


# speedup vs baseline: 1.4480x; 1.4480x over previous
import jax
import jax.numpy as jnp
from jax import lax
from jax.experimental import pallas as pl
from jax.experimental.pallas import tpu as pltpu
from jax.experimental.pallas import tpu_sc as plsc

NC = 2
NS = 16
LANES = 16
NW = NC * NS


def _mesh():
    return plsc.VectorSubcoreMesh(
        core_axis_name="c", subcore_axis_name="s", num_cores=NC, num_subcores=NS
    )


def _sc_gather(table, idx):
    V, D = table.shape
    B = idx.shape[0]
    rows_per = B // NW

    def body(tab_hbm, idx_hbm, out_hbm, idx_v, rows_v, sem):
        cid = lax.axis_index("c")
        sid = lax.axis_index("s")
        wid = cid * NS + sid
        base = wid * rows_per
        pltpu.sync_copy(idx_hbm.at[pl.ds(base, rows_per)], idx_v)
        pltpu.async_copy(tab_hbm.at[idx_v], rows_v, sem).wait()
        pltpu.sync_copy(rows_v, out_hbm.at[pl.ds(base, rows_per)])

    k = pl.kernel(
        body,
        out_type=jax.ShapeDtypeStruct((B, D), table.dtype),
        mesh=_mesh(),
        scratch_types=[
            pltpu.VMEM((rows_per,), jnp.int32),
            pltpu.VMEM((rows_per, D), jnp.float32),
            pltpu.SemaphoreType.DMA,
        ],
    )
    return k(table, idx)


def _sc_gather2(h, c, hi):
    Lv, D = h.shape
    B = hi.shape[0]
    rows_per = B // NW
    GCH = 64

    def body(h_hbm, c_hbm, hi_hbm, hch_hbm, cch_hbm, idx_v, bufa_v, bufb_v,
             sema, semb):
        cid = lax.axis_index("c")
        sid = lax.axis_index("s")
        base = (cid * NS + sid) * rows_per
        wa = wb = None
        for k in range(rows_per // GCH):
            b = base + k * GCH
            pltpu.sync_copy(hi_hbm.at[pl.ds(b, GCH)], idx_v)
            if wa is not None:
                wa.wait()
            ga = pltpu.async_copy(h_hbm.at[idx_v], bufa_v, sema)
            if wb is not None:
                wb.wait()
            gb = pltpu.async_copy(c_hbm.at[idx_v], bufb_v, semb)
            ga.wait()
            wa = pltpu.async_copy(bufa_v, hch_hbm.at[pl.ds(b, GCH)], sema)
            gb.wait()
            wb = pltpu.async_copy(bufb_v, cch_hbm.at[pl.ds(b, GCH)], semb)
        wa.wait()
        wb.wait()

    k = pl.kernel(
        body,
        out_type=[
            jax.ShapeDtypeStruct((B, D), jnp.float32),
            jax.ShapeDtypeStruct((B, D), jnp.float32),
        ],
        mesh=_mesh(),
        scratch_types=[
            pltpu.VMEM((GCH,), jnp.int32),
            pltpu.VMEM((GCH, D), jnp.float32),
            pltpu.VMEM((GCH, D), jnp.float32),
            pltpu.SemaphoreType.DMA,
            pltpu.SemaphoreType.DMA,
        ],
    )
    return k(h, c, hi)


def _sc_segsum(rows, ti, bounds, Lv):
    B, D = rows.shape
    NQ = 8
    Q = Lv // NQ
    CG = D // 4
    ACH = 320

    def body(rows_hbm, ti_hbm, bnd_hbm, out_hbm, acc_v, colbuf0_v, colbuf1_v,
             ti0_v, ti1_v, bnd_v, sem0, sem1):
        cid = lax.axis_index("c")
        sid = lax.axis_index("s")
        wid = cid * NS + sid
        pg = wid // 4
        cg = wid % 4
        pltpu.sync_copy(bnd_hbm.at[pl.ds(wid * LANES, LANES)], bnd_v)
        bvec = bnd_v[...]
        lo = bvec[0]
        hi = bvec[1]
        lo8 = (lo // 8) * 8
        nck = jnp.maximum(hi - lo8 + ACH - 1, 0) // ACH
        nck2 = (nck + 1) // 2
        NCB = CG // LANES

        def chunk_start(ck):
            return jnp.minimum(lo8 + ck * ACH, B - ACH)

        def issue(ck, colbuf, tibuf, sem):
            s = chunk_start(ck)
            pltpu.async_copy(
                rows_hbm.at[pl.ds(s, ACH), pl.ds(cg * CG, CG)], colbuf, sem
            )
            pltpu.async_copy(ti_hbm.at[pl.ds(s, ACH)], tibuf, sem)

        def drain(ck, colbuf, tibuf, sem):
            s = chunk_start(ck)
            pltpu.make_async_copy(
                rows_hbm.at[pl.ds(s, ACH), pl.ds(cg * CG, CG)], colbuf, sem
            ).wait()
            pltpu.make_async_copy(ti_hbm.at[pl.ds(s, ACH)], tibuf, sem).wait()

        @pl.when(nck2 > 0)
        def _():
            issue(0, colbuf0_v, ti0_v, sem0)
            issue(1, colbuf1_v, ti1_v, sem1)

        @pl.loop(0, Q)
        def _(i):
            for cb in range(NCB):
                acc_v[pl.ds(i, 1), pl.ds(cb * LANES, LANES)] = jnp.zeros(
                    (1, LANES), jnp.float32
                )

        def flush(p_prev, a):
            for cb in range(NCB):
                acc_v[pl.ds(p_prev - pg * Q, 1), pl.ds(cb * LANES, LANES)] = (
                    a[cb]
                )

        def scan_chunk(ck, colbuf_v, ti_v, carry):
            s = chunk_start(ck)
            s_nom = lo8 + ck * ACH
            jlo = jnp.maximum(lo, s_nom)

            def group_body(jg, carry):
                p_prev = carry[0]
                a = list(carry[1:])
                tvec = ti_v[pl.ds(jg * LANES, LANES)]
                for l in range(LANES):
                    j = jg * LANES + l
                    r = s + j
                    valid = (r >= jlo) & (r < hi)
                    p = tvec[l]
                    fl = valid & (p != p_prev)

                    @pl.when(fl & (p_prev >= 0))
                    def _(p_prev=p_prev, a=tuple(a)):
                        flush(p_prev, a)

                    row = [
                        colbuf_v[pl.ds(j, 1), pl.ds(cb * LANES, LANES)]
                        for cb in range(NCB)
                    ]
                    a = [
                        jnp.where(
                            valid,
                            jnp.where(fl, row[cb], a[cb] + row[cb]),
                            a[cb],
                        )
                        for cb in range(NCB)
                    ]
                    p_prev = jnp.where(valid, p, p_prev)
                return (p_prev, *a)

            return lax.fori_loop(0, ACH // LANES, group_body, carry)

        def pair_body(ck2, carry):
            ck = 2 * ck2
            drain(ck, colbuf0_v, ti0_v, sem0)
            carry = scan_chunk(ck, colbuf0_v, ti0_v, carry)

            @pl.when(ck + 2 < 2 * nck2)
            def _():
                issue(ck + 2, colbuf0_v, ti0_v, sem0)

            drain(ck + 1, colbuf1_v, ti1_v, sem1)
            carry = scan_chunk(ck + 1, colbuf1_v, ti1_v, carry)

            @pl.when(ck + 3 < 2 * nck2)
            def _():
                issue(ck + 3, colbuf1_v, ti1_v, sem1)

            return carry

        carry0 = (jnp.int32(-1),) + tuple(
            jnp.zeros((1, LANES), jnp.float32) for _ in range(NCB)
        )
        carry = lax.fori_loop(0, nck2, pair_body, carry0)
        p_last = carry[0]

        @pl.when(p_last >= 0)
        def _():
            flush(p_last, carry[1:])

        pltpu.sync_copy(
            acc_v, out_hbm.at[pl.ds(pg * Q, Q), pl.ds(cg * CG, CG)]
        )

    k = pl.kernel(
        body,
        out_type=jax.ShapeDtypeStruct((Lv, D), jnp.float32),
        mesh=_mesh(),
        scratch_types=[
            pltpu.VMEM((Q, CG), jnp.float32),
            pltpu.VMEM((ACH, CG), jnp.float32),
            pltpu.VMEM((ACH, CG), jnp.float32),
            pltpu.VMEM((ACH,), jnp.int32),
            pltpu.VMEM((ACH,), jnp.int32),
            pltpu.VMEM((LANES,), jnp.int32),
            pltpu.SemaphoreType.DMA,
            pltpu.SemaphoreType.DMA,
        ],
    )
    return k(rows, ti, bounds)


def _seg_bounds(ti, Lv):
    edges = jnp.arange(0, Lv + 1, Lv // 8, dtype=jnp.int32)
    b = jnp.searchsorted(ti, edges).astype(jnp.int32)
    pgv = jnp.arange(NW, dtype=jnp.int32) // 4
    row = jnp.zeros((NW, LANES), jnp.int32)
    row = row.at[:, 0].set(b[pgv]).at[:, 1].set(b[pgv + 1])
    return row.reshape(-1)


def _tc_leaf(x0, W_iou, b_iou):
    M, K = x0.shape
    N = W_iou.shape[1]
    H = N // 3
    BM = 512

    def body(x_ref, w_ref, b_ref, h_ref, c_ref):
        iou = jnp.dot(x_ref[...].astype(jnp.bfloat16),
                      w_ref[...].astype(jnp.bfloat16),
                      preferred_element_type=jnp.float32)
        iou = iou + b_ref[...]
        i = jax.nn.sigmoid(iou[:, :H])
        o = jax.nn.sigmoid(iou[:, H:2 * H])
        u = jnp.tanh(iou[:, 2 * H:])
        cc = i * u
        c_ref[...] = cc
        h_ref[...] = o * jnp.tanh(cc)

    return pl.pallas_call(
        body,
        grid=(M // BM,),
        in_specs=[
            pl.BlockSpec((BM, K), lambda i: (i, 0)),
            pl.BlockSpec((K, N), lambda i: (0, 0)),
            pl.BlockSpec((1, N), lambda i: (0, 0)),
        ],
        out_specs=[
            pl.BlockSpec((BM, H), lambda i: (i, 0)),
            pl.BlockSpec((BM, H), lambda i: (i, 0)),
        ],
        out_shape=[
            jax.ShapeDtypeStruct((M, H), jnp.float32),
            jax.ShapeDtypeStruct((M, H), jnp.float32),
        ],
    )(x0, W_iou, b_iou.reshape(1, N))


def _tc_iou(hsum, U_iou, b_iou):
    M, K = hsum.shape
    N = U_iou.shape[1]
    H = N // 3
    BM = 512

    def body(hs_ref, uiou_ref, biou_ref, iu_ref, o_ref):
        hs = hs_ref[...].astype(jnp.bfloat16)
        iou = jnp.dot(hs, uiou_ref[...].astype(jnp.bfloat16),
                      preferred_element_type=jnp.float32)
        iou = iou + biou_ref[...]
        i = jax.nn.sigmoid(iou[:, :H])
        o = jax.nn.sigmoid(iou[:, H:2 * H])
        u = jnp.tanh(iou[:, 2 * H:])
        iu_ref[...] = i * u
        o_ref[...] = o

    return pl.pallas_call(
        body,
        grid=(M // BM,),
        in_specs=[
            pl.BlockSpec((BM, K), lambda i: (i, 0)),
            pl.BlockSpec((K, N), lambda i: (0, 0)),
            pl.BlockSpec((1, N), lambda i: (0, 0)),
        ],
        out_specs=[
            pl.BlockSpec((BM, H), lambda i: (i, 0)),
            pl.BlockSpec((BM, H), lambda i: (i, 0)),
        ],
        out_shape=[
            jax.ShapeDtypeStruct((M, H), jnp.float32),
            jax.ShapeDtypeStruct((M, H), jnp.float32),
        ],
    )(hsum, U_iou, b_iou.reshape(1, N))


def _tc_f(h_ch, c_ch, U_f, b_f):
    B, K = h_ch.shape
    H = U_f.shape[1]
    BM = 512

    def body(hch_ref, cch_ref, uf_ref, bf_ref, g_ref):
        f = jax.nn.sigmoid(
            jnp.dot(hch_ref[...].astype(jnp.bfloat16),
                    uf_ref[...].astype(jnp.bfloat16),
                    preferred_element_type=jnp.float32)
            + bf_ref[...]
        )
        g_ref[...] = f * cch_ref[...]

    return pl.pallas_call(
        body,
        grid=(B // BM,),
        in_specs=[
            pl.BlockSpec((BM, K), lambda i: (i, 0)),
            pl.BlockSpec((BM, K), lambda i: (i, 0)),
            pl.BlockSpec((K, H), lambda i: (0, 0)),
            pl.BlockSpec((1, H), lambda i: (0, 0)),
        ],
        out_specs=pl.BlockSpec((BM, H), lambda i: (i, 0)),
        out_shape=jax.ShapeDtypeStruct((B, H), jnp.float32),
    )(h_ch, c_ch, U_f, b_f.reshape(1, H))


def _tc_final(iu, o, fc):
    M, H = iu.shape
    BM = 1024

    def body(iu_ref, o_ref, fc_ref, h_ref, c_ref):
        c = iu_ref[...] + fc_ref[...]
        c_ref[...] = c
        h_ref[...] = o_ref[...] * jnp.tanh(c)

    return pl.pallas_call(
        body,
        grid=(M // BM,),
        in_specs=[
            pl.BlockSpec((BM, H), lambda i: (i, 0)),
            pl.BlockSpec((BM, H), lambda i: (i, 0)),
            pl.BlockSpec((BM, H), lambda i: (i, 0)),
        ],
        out_specs=[
            pl.BlockSpec((BM, H), lambda i: (i, 0)),
            pl.BlockSpec((BM, H), lambda i: (i, 0)),
        ],
        out_shape=[
            jax.ShapeDtypeStruct((M, H), jnp.float32),
            jax.ShapeDtypeStruct((M, H), jnp.float32),
        ],
    )(iu, o, fc)


def kernel(x, word_idx, hidden_idx, tree_idx, W_iou, U_iou, b_iou, W_f, U_f, b_f):
    L = word_idx.shape[0]
    x0 = _sc_gather(x, word_idx)
    h, c = _tc_leaf(x0, W_iou, b_iou)
    for d in range(hidden_idx.shape[0]):
        hi = hidden_idx[d]
        ti = tree_idx[d]
        bounds = _seg_bounds(ti, L)
        h_ch, c_ch = _sc_gather2(h, c, hi)
        g = _tc_f(h_ch, c_ch, U_f, b_f)
        hsum = _sc_segsum(h_ch, ti, bounds, L)
        fc = _sc_segsum(g, ti, bounds, L)
        iu, o = _tc_iou(hsum, U_iou, b_iou)
        h, c = _tc_final(iu, o, fc)
    return (h, c)


# speedup vs baseline: 1.4595x; 1.0079x over previous
import jax
import jax.numpy as jnp
from jax import lax
from jax.experimental import pallas as pl
from jax.experimental.pallas import tpu as pltpu
from jax.experimental.pallas import tpu_sc as plsc

NC = 2
NS = 16
LANES = 16
NW = NC * NS


def _mesh():
    return plsc.VectorSubcoreMesh(
        core_axis_name="c", subcore_axis_name="s", num_cores=NC, num_subcores=NS
    )


def _sc_gather(table, idx):
    V, D = table.shape
    B = idx.shape[0]
    rows_per = B // NW

    def body(tab_hbm, idx_hbm, out_hbm, idx_v, rows_v, sem):
        cid = lax.axis_index("c")
        sid = lax.axis_index("s")
        wid = cid * NS + sid
        base = wid * rows_per
        pltpu.sync_copy(idx_hbm.at[pl.ds(base, rows_per)], idx_v)
        pltpu.async_copy(tab_hbm.at[idx_v], rows_v, sem).wait()
        pltpu.sync_copy(rows_v, out_hbm.at[pl.ds(base, rows_per)])

    k = pl.kernel(
        body,
        out_type=jax.ShapeDtypeStruct((B, D), table.dtype),
        mesh=_mesh(),
        scratch_types=[
            pltpu.VMEM((rows_per,), jnp.int32),
            pltpu.VMEM((rows_per, D), jnp.float32),
            pltpu.SemaphoreType.DMA,
        ],
    )
    return k(table, idx)


def _sc_gather2(h, c, hi):
    Lv, D = h.shape
    B = hi.shape[0]
    rows_per = B // NW
    GCH = 64

    def body(h_hbm, c_hbm, hi_hbm, hch_hbm, cch_hbm, idx_v, bufa_v, bufb_v,
             sema, semb):
        cid = lax.axis_index("c")
        sid = lax.axis_index("s")
        base = (cid * NS + sid) * rows_per
        wa = wb = None
        for k in range(rows_per // GCH):
            b = base + k * GCH
            pltpu.sync_copy(hi_hbm.at[pl.ds(b, GCH)], idx_v)
            if wa is not None:
                wa.wait()
            ga = pltpu.async_copy(h_hbm.at[idx_v], bufa_v, sema)
            if wb is not None:
                wb.wait()
            gb = pltpu.async_copy(c_hbm.at[idx_v], bufb_v, semb)
            ga.wait()
            wa = pltpu.async_copy(bufa_v, hch_hbm.at[pl.ds(b, GCH)], sema)
            gb.wait()
            wb = pltpu.async_copy(bufb_v, cch_hbm.at[pl.ds(b, GCH)], semb)
        wa.wait()
        wb.wait()

    k = pl.kernel(
        body,
        out_type=[
            jax.ShapeDtypeStruct((B, D), jnp.float32),
            jax.ShapeDtypeStruct((B, D), jnp.float32),
        ],
        mesh=_mesh(),
        scratch_types=[
            pltpu.VMEM((GCH,), jnp.int32),
            pltpu.VMEM((GCH, D), jnp.float32),
            pltpu.VMEM((GCH, D), jnp.float32),
            pltpu.SemaphoreType.DMA,
            pltpu.SemaphoreType.DMA,
        ],
    )
    return k(h, c, hi)


def _sc_segsum(rows, ti, bounds, Lv):
    B, D = rows.shape
    NQ = 8
    Q = Lv // NQ
    CG = D // 4
    ACH = 320

    def body(rows_hbm, ti_hbm, bnd_hbm, out_hbm, acc_v, colbuf0_v, colbuf1_v,
             ti0_v, ti1_v, bnd_v, sem0, sem1):
        cid = lax.axis_index("c")
        sid = lax.axis_index("s")
        wid = cid * NS + sid
        pg = wid // 4
        cg = wid % 4
        pltpu.sync_copy(bnd_hbm.at[pl.ds(wid * LANES, LANES)], bnd_v)
        bvec = bnd_v[...]
        lo = bvec[0]
        hi = bvec[1]
        lo8 = (lo // 8) * 8
        nck = jnp.maximum(hi - lo8 + ACH - 1, 0) // ACH
        nck2 = (nck + 1) // 2
        NCB = CG // LANES

        def chunk_start(ck):
            return jnp.minimum(lo8 + ck * ACH, B - ACH)

        def issue(ck, colbuf, tibuf, sem):
            s = chunk_start(ck)
            pltpu.async_copy(
                rows_hbm.at[pl.ds(s, ACH), pl.ds(cg * CG, CG)], colbuf, sem
            )
            pltpu.async_copy(ti_hbm.at[pl.ds(s, ACH)], tibuf, sem)

        def drain(ck, colbuf, tibuf, sem):
            s = chunk_start(ck)
            pltpu.make_async_copy(
                rows_hbm.at[pl.ds(s, ACH), pl.ds(cg * CG, CG)], colbuf, sem
            ).wait()
            pltpu.make_async_copy(ti_hbm.at[pl.ds(s, ACH)], tibuf, sem).wait()

        @pl.when(nck2 > 0)
        def _():
            issue(0, colbuf0_v, ti0_v, sem0)
            issue(1, colbuf1_v, ti1_v, sem1)

        @pl.loop(0, Q)
        def _(i):
            for cb in range(NCB):
                acc_v[pl.ds(i, 1), pl.ds(cb * LANES, LANES)] = jnp.zeros(
                    (1, LANES), jnp.float32
                )

        def flush(p_prev, a):
            for cb in range(NCB):
                acc_v[pl.ds(p_prev - pg * Q, 1), pl.ds(cb * LANES, LANES)] = (
                    a[cb]
                )

        def scan_chunk(ck, colbuf_v, ti_v, carry):
            s = chunk_start(ck)
            s_nom = lo8 + ck * ACH
            jlo = jnp.maximum(lo, s_nom)

            def group_body(jg, carry):
                p_prev = carry[0]
                a = list(carry[1:])
                tvec = ti_v[pl.ds(jg * LANES, LANES)]
                for l in range(LANES):
                    j = jg * LANES + l
                    r = s + j
                    valid = (r >= jlo) & (r < hi)
                    p = tvec[l]
                    fl = valid & (p != p_prev)

                    @pl.when(fl & (p_prev >= 0))
                    def _(p_prev=p_prev, a=tuple(a)):
                        flush(p_prev, a)

                    row = [
                        colbuf_v[pl.ds(j, 1), pl.ds(cb * LANES, LANES)]
                        for cb in range(NCB)
                    ]
                    a = [
                        jnp.where(
                            valid,
                            jnp.where(fl, row[cb], a[cb] + row[cb]),
                            a[cb],
                        )
                        for cb in range(NCB)
                    ]
                    p_prev = jnp.where(valid, p, p_prev)
                return (p_prev, *a)

            return lax.fori_loop(0, ACH // LANES, group_body, carry)

        def pair_body(ck2, carry):
            ck = 2 * ck2
            drain(ck, colbuf0_v, ti0_v, sem0)
            carry = scan_chunk(ck, colbuf0_v, ti0_v, carry)

            @pl.when(ck + 2 < 2 * nck2)
            def _():
                issue(ck + 2, colbuf0_v, ti0_v, sem0)

            drain(ck + 1, colbuf1_v, ti1_v, sem1)
            carry = scan_chunk(ck + 1, colbuf1_v, ti1_v, carry)

            @pl.when(ck + 3 < 2 * nck2)
            def _():
                issue(ck + 3, colbuf1_v, ti1_v, sem1)

            return carry

        carry0 = (jnp.int32(-1),) + tuple(
            jnp.zeros((1, LANES), jnp.float32) for _ in range(NCB)
        )
        carry = lax.fori_loop(0, nck2, pair_body, carry0)
        p_last = carry[0]

        @pl.when(p_last >= 0)
        def _():
            flush(p_last, carry[1:])

        pltpu.sync_copy(
            acc_v, out_hbm.at[pl.ds(pg * Q, Q), pl.ds(cg * CG, CG)]
        )

    k = pl.kernel(
        body,
        out_type=jax.ShapeDtypeStruct((Lv, D), jnp.float32),
        mesh=_mesh(),
        scratch_types=[
            pltpu.VMEM((Q, CG), jnp.float32),
            pltpu.VMEM((ACH, CG), jnp.float32),
            pltpu.VMEM((ACH, CG), jnp.float32),
            pltpu.VMEM((ACH,), jnp.int32),
            pltpu.VMEM((ACH,), jnp.int32),
            pltpu.VMEM((LANES,), jnp.int32),
            pltpu.SemaphoreType.DMA,
            pltpu.SemaphoreType.DMA,
        ],
    )
    return k(rows, ti, bounds)


def _seg_bounds(ti, Lv):
    edges = jnp.arange(0, Lv + 1, Lv // 8, dtype=jnp.int32)
    b = jnp.searchsorted(ti, edges).astype(jnp.int32)
    pgv = jnp.arange(NW, dtype=jnp.int32) // 4
    row = jnp.zeros((NW, LANES), jnp.int32)
    row = row.at[:, 0].set(b[pgv]).at[:, 1].set(b[pgv + 1])
    return row.reshape(-1)


def _tc_leaf(x0, W_iou, b_iou):
    M, K = x0.shape
    N = W_iou.shape[1]
    H = N // 3
    BM = 1024

    def body(x_ref, w_ref, b_ref, h_ref, c_ref):
        iou = jnp.dot(x_ref[...].astype(jnp.bfloat16),
                      w_ref[...].astype(jnp.bfloat16),
                      preferred_element_type=jnp.float32)
        iou = iou + b_ref[...]
        i = jax.nn.sigmoid(iou[:, :H])
        o = jax.nn.sigmoid(iou[:, H:2 * H])
        u = jnp.tanh(iou[:, 2 * H:])
        cc = i * u
        c_ref[...] = cc
        h_ref[...] = o * jnp.tanh(cc)

    return pl.pallas_call(
        body,
        grid=(M // BM,),
        in_specs=[
            pl.BlockSpec((BM, K), lambda i: (i, 0)),
            pl.BlockSpec((K, N), lambda i: (0, 0)),
            pl.BlockSpec((1, N), lambda i: (0, 0)),
        ],
        out_specs=[
            pl.BlockSpec((BM, H), lambda i: (i, 0)),
            pl.BlockSpec((BM, H), lambda i: (i, 0)),
        ],
        out_shape=[
            jax.ShapeDtypeStruct((M, H), jnp.float32),
            jax.ShapeDtypeStruct((M, H), jnp.float32),
        ],
    )(x0, W_iou, b_iou.reshape(1, N))


def _tc_iou(hsum, U_iou, b_iou):
    M, K = hsum.shape
    N = U_iou.shape[1]
    H = N // 3
    BM = 1024

    def body(hs_ref, uiou_ref, biou_ref, iu_ref, o_ref):
        hs = hs_ref[...].astype(jnp.bfloat16)
        iou = jnp.dot(hs, uiou_ref[...].astype(jnp.bfloat16),
                      preferred_element_type=jnp.float32)
        iou = iou + biou_ref[...]
        i = jax.nn.sigmoid(iou[:, :H])
        o = jax.nn.sigmoid(iou[:, H:2 * H])
        u = jnp.tanh(iou[:, 2 * H:])
        iu_ref[...] = i * u
        o_ref[...] = o

    return pl.pallas_call(
        body,
        grid=(M // BM,),
        in_specs=[
            pl.BlockSpec((BM, K), lambda i: (i, 0)),
            pl.BlockSpec((K, N), lambda i: (0, 0)),
            pl.BlockSpec((1, N), lambda i: (0, 0)),
        ],
        out_specs=[
            pl.BlockSpec((BM, H), lambda i: (i, 0)),
            pl.BlockSpec((BM, H), lambda i: (i, 0)),
        ],
        out_shape=[
            jax.ShapeDtypeStruct((M, H), jnp.float32),
            jax.ShapeDtypeStruct((M, H), jnp.float32),
        ],
    )(hsum, U_iou, b_iou.reshape(1, N))


def _tc_f(h_ch, c_ch, U_f, b_f):
    B, K = h_ch.shape
    H = U_f.shape[1]
    BM = 1024

    def body(hch_ref, cch_ref, uf_ref, bf_ref, g_ref):
        f = jax.nn.sigmoid(
            jnp.dot(hch_ref[...].astype(jnp.bfloat16),
                    uf_ref[...].astype(jnp.bfloat16),
                    preferred_element_type=jnp.float32)
            + bf_ref[...]
        )
        g_ref[...] = f * cch_ref[...]

    return pl.pallas_call(
        body,
        grid=(B // BM,),
        in_specs=[
            pl.BlockSpec((BM, K), lambda i: (i, 0)),
            pl.BlockSpec((BM, K), lambda i: (i, 0)),
            pl.BlockSpec((K, H), lambda i: (0, 0)),
            pl.BlockSpec((1, H), lambda i: (0, 0)),
        ],
        out_specs=pl.BlockSpec((BM, H), lambda i: (i, 0)),
        out_shape=jax.ShapeDtypeStruct((B, H), jnp.float32),
    )(h_ch, c_ch, U_f, b_f.reshape(1, H))


def _tc_final(iu, o, fc):
    M, H = iu.shape
    BM = 1024

    def body(iu_ref, o_ref, fc_ref, h_ref, c_ref):
        c = iu_ref[...] + fc_ref[...]
        c_ref[...] = c
        h_ref[...] = o_ref[...] * jnp.tanh(c)

    return pl.pallas_call(
        body,
        grid=(M // BM,),
        in_specs=[
            pl.BlockSpec((BM, H), lambda i: (i, 0)),
            pl.BlockSpec((BM, H), lambda i: (i, 0)),
            pl.BlockSpec((BM, H), lambda i: (i, 0)),
        ],
        out_specs=[
            pl.BlockSpec((BM, H), lambda i: (i, 0)),
            pl.BlockSpec((BM, H), lambda i: (i, 0)),
        ],
        out_shape=[
            jax.ShapeDtypeStruct((M, H), jnp.float32),
            jax.ShapeDtypeStruct((M, H), jnp.float32),
        ],
    )(iu, o, fc)


def kernel(x, word_idx, hidden_idx, tree_idx, W_iou, U_iou, b_iou, W_f, U_f, b_f):
    L = word_idx.shape[0]
    x0 = _sc_gather(x, word_idx)
    h, c = _tc_leaf(x0, W_iou, b_iou)
    for d in range(hidden_idx.shape[0]):
        hi = hidden_idx[d]
        ti = tree_idx[d]
        bounds = _seg_bounds(ti, L)
        h_ch, c_ch = _sc_gather2(h, c, hi)
        g = _tc_f(h_ch, c_ch, U_f, b_f)
        hsum = _sc_segsum(h_ch, ti, bounds, L)
        fc = _sc_segsum(g, ti, bounds, L)
        iu, o = _tc_iou(hsum, U_iou, b_iou)
        h, c = _tc_final(iu, o, fc)
    return (h, c)
